# baseline (speedup 1.0000x reference)
import functools

import jax
import jax.numpy as jnp
from jax import lax
from jax.experimental import pallas as pl
from jax.experimental.pallas import tpu as pltpu
from jax.experimental.pallas import tpu_sc as plsc

H = 128
B = 100
BOARD = 99
N = B * (BOARD + 1)
E = 320000
T = 3

NC = 2
NS = 16
NW = NC * NS

CHUNK = 128
ECHUNK = 112
EBCH = 24
ENBLK = 4
ECH_PER_TILE = EBCH * ENBLK
NSLOT = 3
NPAD = 10240
ROWS_PER_TILE = NPAD // NS
LCHUNKS = (N + CHUNK - 1) // CHUNK
LCH_ROUND = -(-LCHUNKS // NW)
LPAD = LCHUNKS * CHUNK


@functools.cache
def _build_sc_lookup():
    mesh = plsc.VectorSubcoreMesh(core_axis_name="c", subcore_axis_name="s",
                                  num_cores=NC)

    @functools.partial(
        pl.kernel,
        out_type=jax.ShapeDtypeStruct((LPAD, H), jnp.float32),
        mesh=mesh,
        scratch_types=[
            pltpu.VMEM((CHUNK,), jnp.int32),
            pltpu.VMEM((CHUNK, H), jnp.float32),
            pltpu.SemaphoreType.DMA,
        ],
    )
    def sc_lookup(emb_hbm, idx_hbm, out_hbm, idx_v, rows_v, sem):
        cid = lax.axis_index("c")
        sid = lax.axis_index("s")
        wid = sid * NC + cid
        for r in range(LCH_ROUND):
            chunk = wid + NW * r

            @pl.when(chunk < LCHUNKS)
            def _():
                pltpu.sync_copy(idx_hbm.at[chunk], idx_v)
                pltpu.async_copy(emb_hbm.at[idx_v], rows_v, sem).wait()
                pltpu.sync_copy(rows_v, out_hbm.at[pl.ds(chunk * CHUNK, CHUNK)])

    return sc_lookup


def _sc_lookup(emb, idx_pad):
    return _build_sc_lookup()(emb, idx_pad)


@functools.cache
def _build_sc_segsum():
    mesh = plsc.VectorSubcoreMesh(core_axis_name="c", subcore_axis_name="s",
                                  num_cores=NC)

    @functools.partial(
        pl.kernel,
        out_type=jax.ShapeDtypeStruct((NC, NPAD, H), jnp.float32),
        mesh=mesh,
        scratch_types=[
            pltpu.VMEM((EBCH, ECHUNK), jnp.int32),
            pltpu.VMEM((EBCH, ECHUNK), jnp.int32),
            pltpu.VMEM((ECHUNK, H), jnp.float32),
            pltpu.VMEM((ECHUNK, H), jnp.float32),
            pltpu.VMEM((ECHUNK, H), jnp.float32),
            pltpu.VMEM_SHARED((NPAD, H), jnp.float32),
            pltpu.SemaphoreType.DMA,
            pltpu.SemaphoreType.DMA,
            pltpu.SemaphoreType.DMA,
            pltpu.SemaphoreType.DMA,
        ],
    )
    def sc_segsum(h_hbm, src_hbm, dst_hbm, zeros_hbm, out_hbm,
                  src_v, dst_v, rows0, rows1, rows2, accum,
                  gsem0, gsem1, gsem2, zsem):
        cid = lax.axis_index("c")
        sid = lax.axis_index("s")
        wid = sid * NC + cid
        rows = (rows0, rows1, rows2)
        gsem = (gsem0, gsem1, gsem2)

        pltpu.sync_copy(src_hbm.at[wid, pl.ds(0, EBCH)], src_v)
        pltpu.sync_copy(dst_hbm.at[wid, pl.ds(0, EBCH)], dst_v)
        pltpu.async_copy(h_hbm.at[src_v.at[0]], rows[0], gsem[0])
        pltpu.async_copy(h_hbm.at[src_v.at[1]], rows[1], gsem[1])

        zchunks = [(k * ECHUNK, ECHUNK)
                   for k in range(ROWS_PER_TILE // ECHUNK)]
        zchunks.append((ROWS_PER_TILE - ROWS_PER_TILE % ECHUNK,
                        ROWS_PER_TILE % ECHUNK))
        pltpu.sync_copy(zeros_hbm.at[pl.ds(0, ECHUNK)], rows2)
        for off, ln in zchunks:
            pltpu.async_copy(
                rows2.at[pl.ds(0, ln)],
                accum.at[pl.ds(sid * ROWS_PER_TILE + off, ln)], zsem)
        for off, ln in zchunks:
            pltpu.make_async_copy(
                rows2.at[pl.ds(0, ln)],
                accum.at[pl.ds(sid * ROWS_PER_TILE + off, ln)], zsem).wait()
        plsc.subcore_barrier()

        for b in range(ENBLK):
            if b > 0:
                pltpu.sync_copy(src_hbm.at[wid, pl.ds(b * EBCH, EBCH)], src_v)
                pltpu.sync_copy(dst_hbm.at[wid, pl.ds(b * EBCH, EBCH)], dst_v)
                pltpu.async_copy(h_hbm.at[src_v.at[0]], rows[0], gsem[0])
                pltpu.async_copy(h_hbm.at[src_v.at[1]], rows[1], gsem[1])

            def body(i, carry):
                for u in range(NSLOT):
                    j = NSLOT * i + u
                    pltpu.make_async_copy(h_hbm.at[src_v.at[j]],
                                          rows[u], gsem[u]).wait()

                    @pl.when(j + 2 < EBCH)
                    def _():
                        pltpu.async_copy(h_hbm.at[src_v.at[j + 2]],
                                         rows[(u + 2) % NSLOT],
                                         gsem[(u + 2) % NSLOT])

                    pltpu.sync_copy(rows[u], accum.at[dst_v.at[j]],
                                    add=True)
                return carry

            lax.fori_loop(0, EBCH // NSLOT, body, 0)
        plsc.subcore_barrier()

        base = sid * ROWS_PER_TILE
        for off, ln in zchunks:
            pltpu.sync_copy(accum.at[pl.ds(base + off, ln)],
                            rows2.at[pl.ds(0, ln)])
            pltpu.sync_copy(rows2.at[pl.ds(0, ln)],
                            out_hbm.at[cid, pl.ds(base + off, ln)])

    return sc_segsum


def _sc_segsum(h, src_pad, dst_pad, zeros_blk):
    return _build_sc_segsum()(h, src_pad, dst_pad, zeros_blk)


BLK = 2000
NB_BLK = BLK // (BOARD + 1)


def _lstm_block(x_ref, s0_ref, s1_ref, h_ref, c_ref, wmsgT_ref, wihT_ref,
                whhT_ref, h_out, c_out):
    s = s0_ref[0] + s1_ref[0]
    m = jnp.dot(s, wmsgT_ref[...], preferred_element_type=jnp.float32)
    wihT = wihT_ref[...]
    gates = (jnp.dot(x_ref[...], wihT[:H], preferred_element_type=jnp.float32)
             + jnp.dot(m, wihT[H:], preferred_element_type=jnp.float32)
             + jnp.dot(h_ref[...], whhT_ref[...], preferred_element_type=jnp.float32))
    gi = gates[:, :H]
    gf = gates[:, H:2 * H]
    gg = gates[:, 2 * H:3 * H]
    go = gates[:, 3 * H:]
    c_new = jax.nn.sigmoid(gf) * c_ref[...] + jax.nn.sigmoid(gi) * jnp.tanh(gg)
    h_out[...] = jax.nn.sigmoid(go) * jnp.tanh(c_new)
    c_out[...] = c_new


def _tc_lstm(x, sc_out, h, c, wmsgT, wihT, whhT):
    row_spec = pl.BlockSpec((BLK, H), lambda i: (i, 0))
    full2 = lambda shape: pl.BlockSpec(shape, lambda i: (0, 0))
    return pl.pallas_call(
        _lstm_block,
        grid=(N // BLK,),
        in_specs=[
            row_spec,
            pl.BlockSpec((1, BLK, H), lambda i: (0, i, 0)),
            pl.BlockSpec((1, BLK, H), lambda i: (1, i, 0)),
            row_spec,
            row_spec,
            full2((H, H)),
            full2((2 * H, 4 * H)),
            full2((H, 4 * H)),
        ],
        out_specs=[row_spec, row_spec],
        out_shape=[jax.ShapeDtypeStruct((N, H), jnp.float32),
                   jax.ShapeDtypeStruct((N, H), jnp.float32)],
    )(x, sc_out, sc_out, h, c, wmsgT, wihT, whhT)


def _gol_sum(v, sel):
    n2 = jnp.sum(v * v, axis=1, keepdims=True)
    inv = 1.0 / (jnp.sqrt(n2) + 1e-8)
    vn = v * inv
    bs = jnp.dot(sel, vn, preferred_element_type=jnp.float32)
    tr = jnp.sum(sel * jnp.transpose(n2 * inv * inv))
    return (jnp.sum(bs * bs) - tr) / (BOARD * (BOARD - 1))


def _final_block(h1_ref, h2_ref, h3_ref, emb_ref, woutT_ref,
                 gol_out, in_out, out_out):
    i = pl.program_id(0)
    h3v = h3_ref[...]
    y = jnp.dot(h3v, woutT_ref[...], preferred_element_type=jnp.float32)
    rows = lax.broadcasted_iota(jnp.int32, (BLK, 1), 0)
    mask0 = (rows % (BOARD + 1)) == 0
    embv = emb_ref[...]
    in_out[...] = jnp.where(mask0, embv, h3v)
    out_out[...] = jnp.where(mask0, embv, y)

    colid = lax.broadcasted_iota(jnp.int32, (NB_BLK, BLK), 1)
    rowid = lax.broadcasted_iota(jnp.int32, (NB_BLK, BLK), 0)
    sel = jnp.where((colid // (BOARD + 1) == rowid)
                    & (colid % (BOARD + 1) != 0), 1.0, 0.0)

    part = ((_gol_sum(h1_ref[...], sel) + _gol_sum(h2_ref[...], sel)
             + _gol_sum(h3v, sel)) / (T * B)
            + _gol_sum(y, sel) / B)

    @pl.when(i == 0)
    def _():
        gol_out[...] = jnp.zeros_like(gol_out)

    gol_out[...] += part


def _tc_final(h1, h2, h3, emb_n, woutT):
    row_spec = pl.BlockSpec((BLK, H), lambda i: (i, 0))
    return pl.pallas_call(
        _final_block,
        grid=(N // BLK,),
        in_specs=[row_spec, row_spec, row_spec, row_spec,
                  pl.BlockSpec((H, H), lambda i: (0, 0))],
        out_specs=[pl.BlockSpec((1, 1), lambda i: (0, 0)), row_spec, row_spec],
        out_shape=[jax.ShapeDtypeStruct((1, 1), jnp.float32),
                   jax.ShapeDtypeStruct((N, H), jnp.float32),
                   jax.ShapeDtypeStruct((N, H), jnp.float32)],
    )(h1, h2, h3, emb_n, woutT)


def kernel(node_id, node_sno, edge_index, fixed_embeddings, W_msg, W_ih, W_hh, W_out):
    emb_n = fixed_embeddings[:N]

    lookup_at = (node_id + (BOARD + 1) * node_sno).astype(jnp.int32)
    idx_pad = jnp.pad(lookup_at, (0, LPAD - N)).reshape(LCHUNKS, CHUNK)
    x_pad = _sc_lookup(fixed_embeddings, idx_pad)
    x = x_pad[:N]

    src = edge_index[0].astype(jnp.int32).reshape(NW, E // NW)
    dst = edge_index[1].astype(jnp.int32).reshape(NW, E // NW)
    pad_e = ECH_PER_TILE * ECHUNK - E // NW
    src_pad = jnp.pad(src, ((0, 0), (0, pad_e))).reshape(NW, ECH_PER_TILE, ECHUNK)
    dst_pad = jnp.pad(dst, ((0, 0), (0, pad_e)),
                      constant_values=N).reshape(NW, ECH_PER_TILE, ECHUNK)
    zeros_blk = jnp.zeros((CHUNK, H), jnp.float32)

    wmsgT = W_msg.T
    wihT = W_ih.T
    whhT = W_hh.T

    h = x
    c = x
    steps = []
    for _ in range(T):
        sc_out = _sc_segsum(h, src_pad, dst_pad, zeros_blk)
        h, c = _tc_lstm(x, sc_out, h, c, wmsgT, wihT, whhT)
        steps.append(h)

    gol_arr, in_final, out_final = _tc_final(steps[0], steps[1], steps[2],
                                             emb_n, W_out.T)
    gol = gol_arr[0, 0]
    step_input = jnp.stack(steps, axis=0)
    return (gol, emb_n, in_final, out_final, step_input)
